# initial kernel scaffold (unmeasured)
import jax
import jax.numpy as jnp
from jax import lax
from jax.experimental import pallas as pl
from jax.experimental.pallas import tpu as pltpu

N_LOCAL_EXPERTS = 4
CAP = 640
BLK = 128


def _peer_id():
    mx = lax.axis_index("x")
    my = lax.axis_index("y")
    mz = lax.axis_index("z")
    return (1 - mx, my, mz)


def _pair_barrier():
    barrier = pltpu.get_barrier_semaphore()
    pl.semaphore_signal(
        barrier, inc=1, device_id=_peer_id(),
        device_id_type=pl.DeviceIdType.MESH,
    )
    pl.semaphore_wait(barrier, 1)


def _exchange(x_shard, assign2d):
    tokens, d = x_shard.shape
    ar, ac = assign2d.shape

    def body(x_ref, a_ref, xp_ref, ap_ref, sems):
        peer = _peer_id()
        _pair_barrier()
        rx = pltpu.make_async_remote_copy(
            src_ref=x_ref, dst_ref=xp_ref,
            send_sem=sems.at[0], recv_sem=sems.at[1],
            device_id=peer, device_id_type=pl.DeviceIdType.MESH,
        )
        ra = pltpu.make_async_remote_copy(
            src_ref=a_ref, dst_ref=ap_ref,
            send_sem=sems.at[2], recv_sem=sems.at[3],
            device_id=peer, device_id_type=pl.DeviceIdType.MESH,
        )
        rx.start()
        ra.start()
        rx.wait()
        ra.wait()

    return pl.pallas_call(
        body,
        out_shape=(
            jax.ShapeDtypeStruct((tokens, d), x_shard.dtype),
            jax.ShapeDtypeStruct((ar, ac), assign2d.dtype),
        ),
        in_specs=[
            pl.BlockSpec(memory_space=pltpu.VMEM),
            pl.BlockSpec(memory_space=pltpu.VMEM),
        ],
        out_specs=(
            pl.BlockSpec(memory_space=pltpu.VMEM),
            pl.BlockSpec(memory_space=pltpu.VMEM),
        ),
        scratch_shapes=[pltpu.SemaphoreType.DMA((4,))],
        compiler_params=pltpu.CompilerParams(collective_id=0),
    )(x_shard, assign2d)


def _grouped_ffn(xg, W1, W2):
    rows, d = xg.shape
    n_e, _, f = W1.shape
    nblk = CAP // BLK

    def body(x_ref, w1_ref, w2_ref, o_ref):
        h = jnp.maximum(
            jnp.dot(x_ref[...], w1_ref[0], preferred_element_type=jnp.float32),
            0.0,
        )
        o_ref[...] = jnp.dot(h, w2_ref[0], preferred_element_type=jnp.float32)

    return pl.pallas_call(
        body,
        grid=(n_e, nblk),
        in_specs=[
            pl.BlockSpec((BLK, d), lambda e, i: (e * nblk + i, 0)),
            pl.BlockSpec((1, d, f), lambda e, i: (e, 0, 0)),
            pl.BlockSpec((1, f, d), lambda e, i: (e, 0, 0)),
        ],
        out_specs=pl.BlockSpec((BLK, d), lambda e, i: (e * nblk + i, 0)),
        out_shape=jax.ShapeDtypeStruct((rows, d), jnp.float32),
    )(xg, W1, W2)


def _combine(mine, theirs):
    t, d = mine.shape

    def body(m_ref, t_ref, o_ref, recv_ref, sems):
        peer = _peer_id()
        _pair_barrier()
        r = pltpu.make_async_remote_copy(
            src_ref=t_ref, dst_ref=recv_ref,
            send_sem=sems.at[0], recv_sem=sems.at[1],
            device_id=peer, device_id_type=pl.DeviceIdType.MESH,
        )
        r.start()
        r.wait()
        o_ref[...] = m_ref[...] + recv_ref[...]

    return pl.pallas_call(
        body,
        out_shape=jax.ShapeDtypeStruct((t, d), jnp.float32),
        in_specs=[
            pl.BlockSpec(memory_space=pltpu.VMEM),
            pl.BlockSpec(memory_space=pltpu.VMEM),
        ],
        out_specs=pl.BlockSpec(memory_space=pltpu.VMEM),
        scratch_shapes=[
            pltpu.VMEM((t, d), jnp.float32),
            pltpu.SemaphoreType.DMA((2,)),
        ],
        compiler_params=pltpu.CompilerParams(collective_id=1),
    )(mine, theirs)


def kernel(x, assign, W1, W2):
    tokens, d = x.shape
    mx = lax.axis_index("x")

    a2d = assign.reshape(16, 128)
    x_peer, a_peer2d = _exchange(x, a2d)
    x_all = jnp.concatenate([x, x_peer], axis=0)
    assign_all = jnp.concatenate([assign, a_peer2d.reshape(-1)])

    n_tok = assign_all.shape[0]
    local_e = assign_all - N_LOCAL_EXPERTS * mx
    valid = (local_e >= 0) & (local_e < N_LOCAL_EXPERTS)
    key = jnp.where(valid, local_e, N_LOCAL_EXPERTS).astype(jnp.int32)
    order = jnp.argsort(key, stable=True)
    key_sorted = key[order]
    counts = jnp.sum(
        key[None, :] == jnp.arange(N_LOCAL_EXPERTS, dtype=jnp.int32)[:, None],
        axis=1,
    ).astype(jnp.int32)
    offsets = jnp.concatenate(
        [jnp.zeros((1,), jnp.int32), jnp.cumsum(counts)[:-1].astype(jnp.int32)]
    )
    ks_c = jnp.clip(key_sorted, 0, N_LOCAL_EXPERTS - 1)
    rank = jnp.arange(n_tok, dtype=jnp.int32) - offsets[ks_c]
    dest = jnp.where(
        (key_sorted < N_LOCAL_EXPERTS) & (rank < CAP),
        ks_c * CAP + rank,
        N_LOCAL_EXPERTS * CAP,
    )
    xg = (
        jnp.zeros((N_LOCAL_EXPERTS * CAP, d), jnp.float32)
        .at[dest]
        .set(x_all[order], mode="drop")
    )

    yg = _grouped_ffn(xg, W1, W2)

    dest_by_token = jnp.zeros_like(dest).at[order].set(dest)
    part = jnp.where(
        (dest_by_token < N_LOCAL_EXPERTS * CAP)[:, None],
        yg[jnp.clip(dest_by_token, 0, N_LOCAL_EXPERTS * CAP - 1)],
        0.0,
    )

    return _combine(part[:tokens], part[tokens:])


# baseline (device time: 2376014 ns/iter reference)
import jax
import jax.numpy as jnp
from jax import lax
from jax.experimental import pallas as pl
from jax.experimental.pallas import tpu as pltpu

N_LOCAL_EXPERTS = 4
CAP = 640
BLK = 128


def _peer_id():
    mx = lax.axis_index("x")
    my = lax.axis_index("y")
    mz = lax.axis_index("z")
    return (1 - mx, my, mz)


def _pair_barrier():
    barrier = pltpu.get_barrier_semaphore()
    pl.semaphore_signal(
        barrier, inc=1, device_id=_peer_id(),
        device_id_type=pl.DeviceIdType.MESH,
    )
    pl.semaphore_wait(barrier, 1)


def _exchange(x_shard, assign2d):
    tokens, d = x_shard.shape
    ar, ac = assign2d.shape

    def body(x_ref, a_ref, xp_ref, ap_ref, sems):
        peer = _peer_id()
        _pair_barrier()
        rx = pltpu.make_async_remote_copy(
            src_ref=x_ref, dst_ref=xp_ref,
            send_sem=sems.at[0], recv_sem=sems.at[1],
            device_id=peer, device_id_type=pl.DeviceIdType.MESH,
        )
        ra = pltpu.make_async_remote_copy(
            src_ref=a_ref, dst_ref=ap_ref,
            send_sem=sems.at[2], recv_sem=sems.at[3],
            device_id=peer, device_id_type=pl.DeviceIdType.MESH,
        )
        rx.start()
        ra.start()
        rx.wait()
        ra.wait()

    return pl.pallas_call(
        body,
        out_shape=(
            jax.ShapeDtypeStruct((tokens, d), x_shard.dtype),
            jax.ShapeDtypeStruct((ar, ac), assign2d.dtype),
        ),
        in_specs=[
            pl.BlockSpec(memory_space=pltpu.VMEM),
            pl.BlockSpec(memory_space=pltpu.VMEM),
        ],
        out_specs=(
            pl.BlockSpec(memory_space=pltpu.VMEM),
            pl.BlockSpec(memory_space=pltpu.VMEM),
        ),
        scratch_shapes=[pltpu.SemaphoreType.DMA((4,))],
        compiler_params=pltpu.CompilerParams(
            collective_id=0, vmem_limit_bytes=64 * 1024 * 1024
        ),
    )(x_shard, assign2d)


def _grouped_ffn(xg, W1, W2):
    rows, d = xg.shape
    n_e, _, f = W1.shape
    nblk = CAP // BLK

    def body(x_ref, w1_ref, w2_ref, o_ref):
        h = jnp.maximum(
            jnp.dot(x_ref[...], w1_ref[0], preferred_element_type=jnp.float32),
            0.0,
        )
        o_ref[...] = jnp.dot(h, w2_ref[0], preferred_element_type=jnp.float32)

    return pl.pallas_call(
        body,
        grid=(n_e, nblk),
        in_specs=[
            pl.BlockSpec((BLK, d), lambda e, i: (e * nblk + i, 0)),
            pl.BlockSpec((1, d, f), lambda e, i: (e, 0, 0)),
            pl.BlockSpec((1, f, d), lambda e, i: (e, 0, 0)),
        ],
        out_specs=pl.BlockSpec((BLK, d), lambda e, i: (e * nblk + i, 0)),
        out_shape=jax.ShapeDtypeStruct((rows, d), jnp.float32),
        compiler_params=pltpu.CompilerParams(
            vmem_limit_bytes=64 * 1024 * 1024
        ),
    )(xg, W1, W2)


def _combine(mine, theirs):
    t, d = mine.shape

    def body(m_ref, t_ref, o_ref, recv_ref, sems):
        peer = _peer_id()
        _pair_barrier()
        r = pltpu.make_async_remote_copy(
            src_ref=t_ref, dst_ref=recv_ref,
            send_sem=sems.at[0], recv_sem=sems.at[1],
            device_id=peer, device_id_type=pl.DeviceIdType.MESH,
        )
        r.start()
        r.wait()
        o_ref[...] = m_ref[...] + recv_ref[...]

    return pl.pallas_call(
        body,
        out_shape=jax.ShapeDtypeStruct((t, d), jnp.float32),
        in_specs=[
            pl.BlockSpec(memory_space=pltpu.VMEM),
            pl.BlockSpec(memory_space=pltpu.VMEM),
        ],
        out_specs=pl.BlockSpec(memory_space=pltpu.VMEM),
        scratch_shapes=[
            pltpu.VMEM((t, d), jnp.float32),
            pltpu.SemaphoreType.DMA((2,)),
        ],
        compiler_params=pltpu.CompilerParams(
            collective_id=1, vmem_limit_bytes=64 * 1024 * 1024
        ),
    )(mine, theirs)


def kernel(x, assign, W1, W2):
    tokens, d = x.shape
    mx = lax.axis_index("x")

    a2d = assign.reshape(16, 128)
    x_peer, a_peer2d = _exchange(x, a2d)
    x_all = jnp.concatenate([x, x_peer], axis=0)
    assign_all = jnp.concatenate([assign, a_peer2d.reshape(-1)])

    n_tok = assign_all.shape[0]
    local_e = assign_all - N_LOCAL_EXPERTS * mx
    valid = (local_e >= 0) & (local_e < N_LOCAL_EXPERTS)
    key = jnp.where(valid, local_e, N_LOCAL_EXPERTS).astype(jnp.int32)
    order = jnp.argsort(key, stable=True)
    key_sorted = key[order]
    counts = jnp.sum(
        key[None, :] == jnp.arange(N_LOCAL_EXPERTS, dtype=jnp.int32)[:, None],
        axis=1,
    ).astype(jnp.int32)
    offsets = jnp.concatenate(
        [jnp.zeros((1,), jnp.int32), jnp.cumsum(counts)[:-1].astype(jnp.int32)]
    )
    ks_c = jnp.clip(key_sorted, 0, N_LOCAL_EXPERTS - 1)
    rank = jnp.arange(n_tok, dtype=jnp.int32) - offsets[ks_c]
    dest = jnp.where(
        (key_sorted < N_LOCAL_EXPERTS) & (rank < CAP),
        ks_c * CAP + rank,
        N_LOCAL_EXPERTS * CAP,
    )
    xg = (
        jnp.zeros((N_LOCAL_EXPERTS * CAP, d), jnp.float32)
        .at[dest]
        .set(x_all[order], mode="drop")
    )

    yg = _grouped_ffn(xg, W1, W2)

    dest_by_token = jnp.zeros_like(dest).at[order].set(dest)
    part = jnp.where(
        (dest_by_token < N_LOCAL_EXPERTS * CAP)[:, None],
        yg[jnp.clip(dest_by_token, 0, N_LOCAL_EXPERTS * CAP - 1)],
        0.0,
    )

    return _combine(part[:tokens], part[tokens:])


# device time: 433026 ns/iter; 5.4870x vs baseline; 5.4870x over previous
import jax
import jax.numpy as jnp
from jax import lax
from jax.experimental import pallas as pl
from jax.experimental.pallas import tpu as pltpu

N_LOCAL_EXPERTS = 4
BLK = 256


def _peer_id():
    mx = lax.axis_index("x")
    my = lax.axis_index("y")
    mz = lax.axis_index("z")
    return (1 - mx, my, mz)


def _pair_barrier():
    barrier = pltpu.get_barrier_semaphore()
    pl.semaphore_signal(
        barrier, inc=1, device_id=_peer_id(),
        device_id_type=pl.DeviceIdType.MESH,
    )
    pl.semaphore_wait(barrier, 1)


def _exchange(x_shard, assign2d):
    tokens, d = x_shard.shape
    ar, ac = assign2d.shape

    def body(x_ref, a_ref, xp_ref, ap_ref, sems):
        peer = _peer_id()
        _pair_barrier()
        rx = pltpu.make_async_remote_copy(
            src_ref=x_ref, dst_ref=xp_ref,
            send_sem=sems.at[0], recv_sem=sems.at[1],
            device_id=peer, device_id_type=pl.DeviceIdType.MESH,
        )
        ra = pltpu.make_async_remote_copy(
            src_ref=a_ref, dst_ref=ap_ref,
            send_sem=sems.at[2], recv_sem=sems.at[3],
            device_id=peer, device_id_type=pl.DeviceIdType.MESH,
        )
        rx.start()
        ra.start()
        rx.wait()
        ra.wait()

    return pl.pallas_call(
        body,
        out_shape=(
            jax.ShapeDtypeStruct((tokens, d), x_shard.dtype),
            jax.ShapeDtypeStruct((ar, ac), assign2d.dtype),
        ),
        in_specs=[
            pl.BlockSpec(memory_space=pltpu.VMEM),
            pl.BlockSpec(memory_space=pltpu.VMEM),
        ],
        out_specs=(
            pl.BlockSpec(memory_space=pltpu.VMEM),
            pl.BlockSpec(memory_space=pltpu.VMEM),
        ),
        scratch_shapes=[pltpu.SemaphoreType.DMA((4,))],
        compiler_params=pltpu.CompilerParams(
            collective_id=0, vmem_limit_bytes=64 * 1024 * 1024
        ),
    )(x_shard, assign2d)


def _dense_moe(x_all, mask, W1, W2):
    n_tok, d = x_all.shape
    n_e, _, f = W1.shape
    nblk = n_tok // BLK

    def body(x_ref, m_ref, w1_ref, w2_ref, o_ref):
        e = pl.program_id(0)
        i = pl.program_id(1)
        h = jnp.maximum(
            jnp.dot(x_ref[...], w1_ref[0], preferred_element_type=jnp.float32),
            0.0,
        )
        y = jnp.dot(h, w2_ref[0], preferred_element_type=jnp.float32)
        y = y * m_ref[0][:, None]
        rows = pl.ds(i * BLK, BLK)

        @pl.when(e == 0)
        def _():
            o_ref[rows, :] = y

        @pl.when(e != 0)
        def _():
            o_ref[rows, :] += y

    return pl.pallas_call(
        body,
        grid=(n_e, nblk),
        in_specs=[
            pl.BlockSpec((BLK, d), lambda e, i: (i, 0)),
            pl.BlockSpec((8, BLK), lambda e, i: (e, i)),
            pl.BlockSpec((1, d, f), lambda e, i: (e, 0, 0)),
            pl.BlockSpec((1, f, d), lambda e, i: (e, 0, 0)),
        ],
        out_specs=pl.BlockSpec((n_tok, d), lambda e, i: (0, 0)),
        out_shape=jax.ShapeDtypeStruct((n_tok, d), jnp.float32),
        compiler_params=pltpu.CompilerParams(
            vmem_limit_bytes=100 * 1024 * 1024
        ),
    )(x_all, mask, W1, W2)


def _combine(mine, theirs):
    t, d = mine.shape

    def body(m_ref, t_ref, o_ref, recv_ref, sems):
        peer = _peer_id()
        _pair_barrier()
        r = pltpu.make_async_remote_copy(
            src_ref=t_ref, dst_ref=recv_ref,
            send_sem=sems.at[0], recv_sem=sems.at[1],
            device_id=peer, device_id_type=pl.DeviceIdType.MESH,
        )
        r.start()
        r.wait()
        o_ref[...] = m_ref[...] + recv_ref[...]

    return pl.pallas_call(
        body,
        out_shape=jax.ShapeDtypeStruct((t, d), jnp.float32),
        in_specs=[
            pl.BlockSpec(memory_space=pltpu.VMEM),
            pl.BlockSpec(memory_space=pltpu.VMEM),
        ],
        out_specs=pl.BlockSpec(memory_space=pltpu.VMEM),
        scratch_shapes=[
            pltpu.VMEM((t, d), jnp.float32),
            pltpu.SemaphoreType.DMA((2,)),
        ],
        compiler_params=pltpu.CompilerParams(
            collective_id=1, vmem_limit_bytes=64 * 1024 * 1024
        ),
    )(mine, theirs)


def kernel(x, assign, W1, W2):
    tokens, d = x.shape
    mx = lax.axis_index("x")

    x_peer, a_peer2d = _exchange(x, assign.reshape(16, 128))
    x_all = jnp.concatenate([x, x_peer], axis=0)
    assign_all = jnp.concatenate([assign, a_peer2d.reshape(-1)])

    e_ids = N_LOCAL_EXPERTS * mx + jnp.arange(N_LOCAL_EXPERTS, dtype=jnp.int32)
    mask = (assign_all[None, :] == e_ids[:, None]).astype(jnp.float32)
    mask8 = jnp.broadcast_to(
        mask[:, None, :], (N_LOCAL_EXPERTS, 8, mask.shape[1])
    ).reshape(N_LOCAL_EXPERTS * 8, mask.shape[1])
    part = _dense_moe(x_all, mask8, W1, W2)

    return _combine(part[:tokens], part[tokens:])


# device time: 217386 ns/iter; 10.9299x vs baseline; 1.9920x over previous
import jax
import jax.numpy as jnp
from jax import lax
from jax.experimental import pallas as pl
from jax.experimental.pallas import tpu as pltpu

N_LOCAL_EXPERTS = 4
BLK = 256
OH_LANES = 128


def _fused_moe_a2a(x16, oh_keep, oh_send, W1_16, W2_16):
    tokens, d = x16.shape
    n_e, _, f = W1_16.shape
    nblk = tokens // BLK

    def body(x_ref, ohk_ref, ohs_ref, w1_ref, w2_ref, out_ref,
             xp_ref, ohr_ref, ppart_ref, rpart_ref,
             in_sems, blk_send_sems, blk_recv_sems):
        mx = lax.axis_index("x")
        my = lax.axis_index("y")
        mz = lax.axis_index("z")
        peer = (1 - mx, my, mz)

        barrier = pltpu.get_barrier_semaphore()
        pl.semaphore_signal(
            barrier, inc=1, device_id=peer,
            device_id_type=pl.DeviceIdType.MESH,
        )
        pl.semaphore_wait(barrier, 1)

        rdma_x = pltpu.make_async_remote_copy(
            src_ref=x_ref, dst_ref=xp_ref,
            send_sem=in_sems.at[0], recv_sem=in_sems.at[1],
            device_id=peer, device_id_type=pl.DeviceIdType.MESH,
        )
        rdma_oh = pltpu.make_async_remote_copy(
            src_ref=ohs_ref, dst_ref=ohr_ref,
            send_sem=in_sems.at[2], recv_sem=in_sems.at[3],
            device_id=peer, device_id_type=pl.DeviceIdType.MESH,
        )
        rdma_x.start()
        rdma_oh.start()

        def ffn_block(xb, oh_ref_, rows):
            acc = jnp.zeros((BLK, d), jnp.float32)
            for e in range(n_e):
                h = jnp.maximum(
                    jnp.dot(xb, w1_ref[e], preferred_element_type=jnp.float32),
                    0.0,
                ).astype(jnp.bfloat16)
                y = jnp.dot(h, w2_ref[e], preferred_element_type=jnp.float32)
                acc = acc + y * oh_ref_[rows, e:e + 1]
            return acc

        def my_blk(i, _):
            rows = pl.ds(i * BLK, BLK)
            out_ref[rows, :] = ffn_block(x_ref[rows, :], ohk_ref, rows)
            return 0

        lax.fori_loop(0, nblk, my_blk, 0)

        rdma_x.wait()
        rdma_oh.wait()

        def peer_blk(i, _):
            rows = pl.ds(i * BLK, BLK)
            ppart_ref[rows, :] = ffn_block(
                xp_ref[rows, :], ohr_ref, rows
            ).astype(jnp.bfloat16)
            send = pltpu.make_async_remote_copy(
                src_ref=ppart_ref.at[rows, :],
                dst_ref=rpart_ref.at[rows, :],
                send_sem=blk_send_sems.at[i],
                recv_sem=blk_recv_sems.at[i],
                device_id=peer, device_id_type=pl.DeviceIdType.MESH,
            )
            send.start()
            return 0

        lax.fori_loop(0, nblk, peer_blk, 0)

        def add_blk(i, _):
            rows = pl.ds(i * BLK, BLK)
            done = pltpu.make_async_remote_copy(
                src_ref=ppart_ref.at[rows, :],
                dst_ref=rpart_ref.at[rows, :],
                send_sem=blk_send_sems.at[i],
                recv_sem=blk_recv_sems.at[i],
                device_id=peer, device_id_type=pl.DeviceIdType.MESH,
            )
            done.wait()
            out_ref[rows, :] += rpart_ref[rows, :].astype(jnp.float32)
            return 0

        lax.fori_loop(0, nblk, add_blk, 0)

    return pl.pallas_call(
        body,
        out_shape=jax.ShapeDtypeStruct((tokens, d), jnp.float32),
        in_specs=[pl.BlockSpec(memory_space=pltpu.VMEM)] * 5,
        out_specs=pl.BlockSpec(memory_space=pltpu.VMEM),
        scratch_shapes=[
            pltpu.VMEM((tokens, d), jnp.bfloat16),
            pltpu.VMEM((tokens, OH_LANES), jnp.float32),
            pltpu.VMEM((tokens, d), jnp.bfloat16),
            pltpu.VMEM((tokens, d), jnp.bfloat16),
            pltpu.SemaphoreType.DMA((4,)),
            pltpu.SemaphoreType.DMA((8,)),
            pltpu.SemaphoreType.DMA((8,)),
        ],
        compiler_params=pltpu.CompilerParams(
            collective_id=0, vmem_limit_bytes=100 * 1024 * 1024
        ),
    )(x16, oh_keep, oh_send, W1_16, W2_16)


def kernel(x, assign, W1, W2):
    mx = lax.axis_index("x")

    e_mine = N_LOCAL_EXPERTS * mx + jnp.arange(OH_LANES, dtype=jnp.int32)
    e_peer = N_LOCAL_EXPERTS * (1 - mx) + jnp.arange(OH_LANES, dtype=jnp.int32)
    oh_keep = (assign[:, None] == e_mine[None, :]).astype(jnp.float32)
    oh_send = (assign[:, None] == e_peer[None, :]).astype(jnp.float32)

    return _fused_moe_a2a(
        x.astype(jnp.bfloat16),
        oh_keep,
        oh_send,
        W1.astype(jnp.bfloat16),
        W2.astype(jnp.bfloat16),
    )
